# baseline (speedup 1.0000x reference)
import functools

import jax
import jax.numpy as jnp
from jax import lax
from jax.experimental import pallas as pl
from jax.experimental.pallas import tpu as pltpu
from jax.experimental.pallas import tpu_sc as plsc

_NC = 2
_NS = 16
_NW = _NC * _NS
_L = 16
_C = 80
_SG = 8
_RNB = 4


def _sc_body(ego_hbm, iv_hbm, val_hbm, out_hbm,
             ivbuf0, ivbuf1, vbuf0, vbuf1, rows0, rows1, rows2, rows3,
             acc_sh, gsems, ssems, isems,
             *, n_chunks, n, d, base_span, piece, rem):
    c = lax.axis_index("c")
    s = lax.axis_index("s")
    wid = c * _NS + s
    rows = (rows0, rows1, rows2, rows3)
    ivbufs = (ivbuf0, ivbuf1)
    vbufs = (vbuf0, vbuf1)
    sgn = n_chunks // _SG
    zero = jnp.zeros((_L,), jnp.float32)

    def zrow(i, carry):
        for j in range(d // _L):
            rows0[i, pl.ds(j * _L, _L)] = zero
        return carry

    lax.fori_loop(0, _C, zrow, 0)

    base_row = s * base_span
    for k in range(base_span // piece):
        pltpu.sync_copy(rows0.at[pl.ds(0, piece)],
                        acc_sh.at[pl.ds(base_row + k * piece, piece)])
    if rem:
        @pl.when(s == _NS - 1)
        def _zero_rem():
            pltpu.sync_copy(rows0.at[pl.ds(0, rem)],
                            acc_sh.at[pl.ds(_NS * base_span, rem)])
    plsc.subcore_barrier()

    def gissue(ivbuf, j, b):
        pltpu.async_copy(ego_hbm.at[ivbuf.at[j, 0]], rows[b], gsems.at[b])

    def inner(sg, h):
        ivbuf = ivbufs[h]
        vbuf = vbufs[h]

        def triple(q, carry):
            for b in range(_RNB):
                j = q * _RNB + b
                pltpu.make_async_copy(ego_hbm.at[ivbuf.at[j, 0]], rows[b],
                                      gsems.at[b]).wait()

                def scale(e16, carry2, b=b, j=j):
                    e0 = e16 * _L
                    vals16 = vbuf[j, 0, pl.ds(e0, _L)]
                    for i in range(_L):
                        v = vals16[i]
                        for jj in range(d // _L):
                            sl = pl.ds(jj * _L, _L)
                            rows[b][e0 + i, sl] = rows[b][e0 + i, sl] * v
                    return carry2

                lax.fori_loop(0, _C // _L, scale, 0)
                pltpu.async_copy(rows[b], acc_sh.at[ivbuf.at[j, 1]],
                                 ssems.at[b], add=True)

                bp = (b + _RNB - 1) % _RNB

                def _swait(bp=bp, j=j):
                    pltpu.make_async_copy(rows[bp], acc_sh.at[ivbuf.at[j, 1]],
                                          ssems.at[bp]).wait()
                if b == 0:
                    pl.when((q > 0) | (sg > 0))(_swait)
                else:
                    _swait()

                @pl.when(q * _RNB + b + (_RNB - 1) < _SG)
                def _queue(q=q, b=b, bp=bp):
                    gissue(ivbuf, q * _RNB + b + (_RNB - 1), bp)
            return carry

        lax.fori_loop(0, _SG // _RNB, triple, 0)

        @pl.when(sg + 1 < sgn)
        def _cross():
            pltpu.make_async_copy(iv_hbm.at[wid, pl.ds(0, _SG)],
                                  ivbufs[1 - h], isems.at[1 - h]).wait()
            pltpu.make_async_copy(val_hbm.at[wid, pl.ds(0, _SG)],
                                  vbufs[1 - h], isems.at[2 + (1 - h)]).wait()
            for m in range(_RNB - 1):
                gissue(ivbufs[1 - h], m, m)

    pltpu.async_copy(iv_hbm.at[wid, pl.ds(0, _SG)], ivbuf0, isems.at[0])
    pltpu.async_copy(val_hbm.at[wid, pl.ds(0, _SG)], vbuf0, isems.at[2])
    pltpu.make_async_copy(iv_hbm.at[wid, pl.ds(0, _SG)], ivbuf0,
                          isems.at[0]).wait()
    pltpu.make_async_copy(val_hbm.at[wid, pl.ds(0, _SG)], vbuf0,
                          isems.at[2]).wait()
    for _m in range(_RNB - 1):
        gissue(ivbuf0, _m, _m)

    def sgpair(p, carry):
        sg0 = p * 2
        for h in range(2):
            sg = sg0 + h

            @pl.when(sg + 1 < sgn)
            def _prefetch(sg=sg, h=h):
                pltpu.async_copy(iv_hbm.at[wid, pl.ds((sg + 1) * _SG, _SG)],
                                 ivbufs[1 - h], isems.at[1 - h])
                pltpu.async_copy(val_hbm.at[wid, pl.ds((sg + 1) * _SG, _SG)],
                                 vbufs[1 - h], isems.at[2 + (1 - h)])
            inner(sg, h)
        return carry

    lax.fori_loop(0, sgn // 2, sgpair, 0)
    _fb = (n_chunks - 1) % _RNB
    pltpu.make_async_copy(rows[_fb], acc_sh.at[ivbuf0.at[0, 1]],
                          ssems.at[_fb]).wait()
    plsc.subcore_barrier()

    pltpu.sync_copy(acc_sh.at[pl.ds(base_row, base_span)],
                    out_hbm.at[c, pl.ds(base_row, base_span)])
    if rem:
        @pl.when(s == _NS - 1)
        def _wb_rem():
            r0 = _NS * base_span
            pltpu.sync_copy(acc_sh.at[pl.ds(r0, rem)],
                            out_hbm.at[c, pl.ds(r0, rem)])


def _sc_spmm(ego, iv, val, n_chunks, n, d):
    base_span = (n // _NS) // 8 * 8
    rem = n - _NS * base_span
    assert rem % 8 == 0 and rem <= _C
    piece = max(p for p in range(8, _C + 1, 8) if base_span % p == 0)
    mesh = plsc.VectorSubcoreMesh(core_axis_name="c", subcore_axis_name="s",
                                  num_cores=_NC, num_subcores=_NS)
    f = pl.kernel(
        functools.partial(_sc_body, n_chunks=n_chunks, n=n, d=d,
                          base_span=base_span, piece=piece, rem=rem),
        out_type=jax.ShapeDtypeStruct((_NC, n, d), jnp.float32),
        mesh=mesh,
        scratch_types=[
            pltpu.VMEM((_SG, 2, _C), jnp.int32),
            pltpu.VMEM((_SG, 2, _C), jnp.int32),
            pltpu.VMEM((_SG, 1, _C), jnp.float32),
            pltpu.VMEM((_SG, 1, _C), jnp.float32),
            pltpu.VMEM((_C, d), jnp.float32),
            pltpu.VMEM((_C, d), jnp.float32),
            pltpu.VMEM((_C, d), jnp.float32),
            pltpu.VMEM((_C, d), jnp.float32),
            pltpu.VMEM_SHARED((n, d), jnp.float32),
            pltpu.SemaphoreType.DMA((_RNB,)),
            pltpu.SemaphoreType.DMA((_RNB,)),
            pltpu.SemaphoreType.DMA((4,)),
        ],
    )
    return f(ego, iv, val)


def _tc_combine(ego, partials, wt, b, g, beta):
    n, d = ego.shape
    blk = 1000

    def body(ego_ref, p_ref, wt_ref, b_ref, g_ref, beta_ref, o_ref):
        hi = ego_ref[...] + p_ref[0] + p_ref[1]
        y = jnp.dot(hi, wt_ref[...], preferred_element_type=jnp.float32)
        y = y + b_ref[...]
        y = jnp.where(y >= 0, y, 0.01 * y)
        m = jnp.mean(y, axis=-1, keepdims=True)
        v = jnp.mean((y - m) ** 2, axis=-1, keepdims=True)
        o_ref[...] = (y - m) * lax.rsqrt(v + 1e-5) * g_ref[...] + beta_ref[...]

    return pl.pallas_call(
        body,
        grid=(n // blk,),
        in_specs=[
            pl.BlockSpec((blk, d), lambda i: (i, 0)),
            pl.BlockSpec((_NC, blk, d), lambda i: (0, i, 0)),
            pl.BlockSpec((d, d), lambda i: (0, 0)),
            pl.BlockSpec((1, d), lambda i: (0, 0)),
            pl.BlockSpec((1, d), lambda i: (0, 0)),
            pl.BlockSpec((1, d), lambda i: (0, 0)),
        ],
        out_specs=pl.BlockSpec((blk, d), lambda i: (i, 0)),
        out_shape=jax.ShapeDtypeStruct((n, d), jnp.float32),
    )(ego, partials, wt, b.reshape(1, d), g.reshape(1, d), beta.reshape(1, d))


def kernel(ego_embeddings, a_in_edge_index, a_in_edge_values, all_layers_0,
           lamda, alpha, l, lin_W, lin_b, ln_g, ln_beta):
    n, d = ego_embeddings.shape
    e = a_in_edge_values.shape[0]
    assert n % _NS == 0 and d % _L == 0

    n_chunks = -(-e // (_NW * _C))
    n_chunks = -(-n_chunks // (2 * _SG)) * (2 * _SG)
    e_pad = n_chunks * _NW * _C
    pad = e_pad - e

    src = a_in_edge_index[0].astype(jnp.int32)
    dst = a_in_edge_index[1].astype(jnp.int32)
    val = a_in_edge_values.astype(jnp.float32)
    if pad:
        src = jnp.concatenate([src, jnp.arange(pad, dtype=jnp.int32) % n])
        dst = jnp.concatenate([dst, jnp.zeros((pad,), jnp.int32)])
        val = jnp.concatenate([val, jnp.zeros((pad,), jnp.float32)])
    src = src.reshape(_NW, n_chunks, _C)
    wids = jnp.arange(_NW, dtype=jnp.int32)[:, None, None]
    src = src + ((wids // _NS) * 2 + (wids % 2)) * n
    iv = jnp.stack([src, dst.reshape(_NW, n_chunks, _C)], axis=2)
    val = val.reshape(_NW, n_chunks, 1, _C)
    ego4 = jnp.concatenate([ego_embeddings] * 4, axis=0)

    partials = _sc_spmm(ego4, iv, val, n_chunks, n, d)
    return _tc_combine(ego_embeddings, partials, lin_W.T, lin_b, ln_g, ln_beta)


# speedup vs baseline: 1.0994x; 1.0994x over previous
import functools

import jax
import jax.numpy as jnp
from jax import lax
from jax.experimental import pallas as pl
from jax.experimental.pallas import tpu as pltpu
from jax.experimental.pallas import tpu_sc as plsc

_NC = 2
_NS = 16
_NW = _NC * _NS
_L = 16
_C = 80
_SG = 8
_RNB = 4


def _sc_body(ego_hbm, iv_hbm, val_hbm, out_hbm,
             ivbuf0, ivbuf1, vbuf0, vbuf1, rows0, rows1, rows2, rows3,
             acc_sh, gsems, ssems, isems,
             *, n_chunks, n, d, base_span, piece, rem):
    c = lax.axis_index("c")
    s = lax.axis_index("s")
    wid = c * _NS + s
    rows = (rows0, rows1, rows2, rows3)
    ivbufs = (ivbuf0, ivbuf1)
    vbufs = (vbuf0, vbuf1)
    sgn = n_chunks // _SG
    zero = jnp.zeros((_L,), jnp.float32)

    def zrow(i, carry):
        for j in range(d // _L):
            rows0[i, pl.ds(j * _L, _L)] = zero
        return carry

    lax.fori_loop(0, _C, zrow, 0)

    base_row = s * base_span
    for k in range(base_span // piece):
        pltpu.sync_copy(rows0.at[pl.ds(0, piece)],
                        acc_sh.at[pl.ds(base_row + k * piece, piece)])
    if rem:
        @pl.when(s == _NS - 1)
        def _zero_rem():
            pltpu.sync_copy(rows0.at[pl.ds(0, rem)],
                            acc_sh.at[pl.ds(_NS * base_span, rem)])
    plsc.subcore_barrier()

    def gissue(ivbuf, j, b):
        pltpu.async_copy(ego_hbm.at[ivbuf.at[j, 0]], rows[b], gsems.at[b])

    def inner(sg, h):
        ivbuf = ivbufs[h]
        vbuf = vbufs[h]

        def triple(q, carry):
            for b in range(_RNB):
                j = q * _RNB + b
                pltpu.make_async_copy(ego_hbm.at[ivbuf.at[j, 0]], rows[b],
                                      gsems.at[b]).wait()

                def scale(e16, carry2, b=b, j=j):
                    e0 = e16 * _L
                    vals16 = vbuf[j, 0, pl.ds(e0, _L)]
                    for i in range(_L):
                        v = vals16[i]
                        for jj in range(d // _L):
                            sl = pl.ds(jj * _L, _L)
                            rows[b][e0 + i, sl] = rows[b][e0 + i, sl] * v
                    return carry2

                lax.fori_loop(0, _C // _L, scale, 0)
                pltpu.async_copy(rows[b], acc_sh.at[ivbuf.at[j, 1]],
                                 ssems.at[b], add=True)

                bp = (b + _RNB - 1) % _RNB

                def _swait(bp=bp, j=j):
                    pltpu.make_async_copy(rows[bp], acc_sh.at[ivbuf.at[j, 1]],
                                          ssems.at[bp]).wait()
                if b == 0:
                    pl.when((q > 0) | (sg > 0))(_swait)
                else:
                    _swait()

                @pl.when(q * _RNB + b + (_RNB - 1) < _SG)
                def _queue(q=q, b=b, bp=bp):
                    gissue(ivbuf, q * _RNB + b + (_RNB - 1), bp)
            return carry

        lax.fori_loop(0, _SG // _RNB, triple, 0)

        @pl.when(sg + 1 < sgn)
        def _cross():
            pltpu.make_async_copy(iv_hbm.at[wid, pl.ds(0, _SG)],
                                  ivbufs[1 - h], isems.at[1 - h]).wait()
            pltpu.make_async_copy(val_hbm.at[wid, pl.ds(0, _SG)],
                                  vbufs[1 - h], isems.at[2 + (1 - h)]).wait()
            for m in range(_RNB - 1):
                gissue(ivbufs[1 - h], m, m)

    pltpu.async_copy(iv_hbm.at[wid, pl.ds(0, _SG)], ivbuf0, isems.at[0])
    pltpu.async_copy(val_hbm.at[wid, pl.ds(0, _SG)], vbuf0, isems.at[2])
    pltpu.make_async_copy(iv_hbm.at[wid, pl.ds(0, _SG)], ivbuf0,
                          isems.at[0]).wait()
    pltpu.make_async_copy(val_hbm.at[wid, pl.ds(0, _SG)], vbuf0,
                          isems.at[2]).wait()
    for _m in range(_RNB - 1):
        gissue(ivbuf0, _m, _m)

    def sgpair(p, carry):
        sg0 = p * 2
        for h in range(2):
            sg = sg0 + h

            @pl.when(sg + 1 < sgn)
            def _prefetch(sg=sg, h=h):
                pltpu.async_copy(iv_hbm.at[wid, pl.ds((sg + 1) * _SG, _SG)],
                                 ivbufs[1 - h], isems.at[1 - h])
                pltpu.async_copy(val_hbm.at[wid, pl.ds((sg + 1) * _SG, _SG)],
                                 vbufs[1 - h], isems.at[2 + (1 - h)])
            inner(sg, h)
        return carry

    lax.fori_loop(0, sgn // 2, sgpair, 0)
    _fb = (n_chunks - 1) % _RNB
    pltpu.make_async_copy(rows[_fb], acc_sh.at[ivbuf0.at[0, 1]],
                          ssems.at[_fb]).wait()
    plsc.subcore_barrier()

    pltpu.sync_copy(acc_sh.at[pl.ds(base_row, base_span)],
                    out_hbm.at[c, pl.ds(base_row, base_span)])
    if rem:
        @pl.when(s == _NS - 1)
        def _wb_rem():
            r0 = _NS * base_span
            pltpu.sync_copy(acc_sh.at[pl.ds(r0, rem)],
                            out_hbm.at[c, pl.ds(r0, rem)])


def _sc_spmm(ego, iv, val, n_chunks, n, d):
    base_span = (n // _NS) // 8 * 8
    rem = n - _NS * base_span
    assert rem % 8 == 0 and rem <= _C
    piece = max(p for p in range(8, _C + 1, 8) if base_span % p == 0)
    mesh = plsc.VectorSubcoreMesh(core_axis_name="c", subcore_axis_name="s",
                                  num_cores=_NC, num_subcores=_NS)
    f = pl.kernel(
        functools.partial(_sc_body, n_chunks=n_chunks, n=n, d=d,
                          base_span=base_span, piece=piece, rem=rem),
        out_type=jax.ShapeDtypeStruct((_NC, n, d), jnp.float32),
        mesh=mesh,
        scratch_types=[
            pltpu.VMEM((_SG, 2, _C), jnp.int32),
            pltpu.VMEM((_SG, 2, _C), jnp.int32),
            pltpu.VMEM((_SG, 1, _C), jnp.float32),
            pltpu.VMEM((_SG, 1, _C), jnp.float32),
            pltpu.VMEM((_C, d), jnp.float32),
            pltpu.VMEM((_C, d), jnp.float32),
            pltpu.VMEM((_C, d), jnp.float32),
            pltpu.VMEM((_C, d), jnp.float32),
            pltpu.VMEM_SHARED((n, d), jnp.float32),
            pltpu.SemaphoreType.DMA((_RNB,)),
            pltpu.SemaphoreType.DMA((_RNB,)),
            pltpu.SemaphoreType.DMA((4,)),
        ],
    )
    return f(ego, iv, val)


def _tc_combine(ego, partials, wt, b, g, beta):
    n, d = ego.shape
    blk = 1000

    def body(ego_ref, p_ref, wt_ref, b_ref, g_ref, beta_ref, o_ref):
        hi = ego_ref[...] + p_ref[0] + p_ref[1]
        y = jnp.dot(hi, wt_ref[...], preferred_element_type=jnp.float32)
        y = y + b_ref[...]
        y = jnp.where(y >= 0, y, 0.01 * y)
        m = jnp.mean(y, axis=-1, keepdims=True)
        v = jnp.mean((y - m) ** 2, axis=-1, keepdims=True)
        o_ref[...] = (y - m) * lax.rsqrt(v + 1e-5) * g_ref[...] + beta_ref[...]

    return pl.pallas_call(
        body,
        grid=(n // blk,),
        in_specs=[
            pl.BlockSpec((blk, d), lambda i: (i, 0)),
            pl.BlockSpec((_NC, blk, d), lambda i: (0, i, 0)),
            pl.BlockSpec((d, d), lambda i: (0, 0)),
            pl.BlockSpec((1, d), lambda i: (0, 0)),
            pl.BlockSpec((1, d), lambda i: (0, 0)),
            pl.BlockSpec((1, d), lambda i: (0, 0)),
        ],
        out_specs=pl.BlockSpec((blk, d), lambda i: (i, 0)),
        out_shape=jax.ShapeDtypeStruct((n, d), jnp.float32),
    )(ego, partials, wt, b.reshape(1, d), g.reshape(1, d), beta.reshape(1, d))


def kernel(ego_embeddings, a_in_edge_index, a_in_edge_values, all_layers_0,
           lamda, alpha, l, lin_W, lin_b, ln_g, ln_beta):
    n, d = ego_embeddings.shape
    e = a_in_edge_values.shape[0]
    assert n % _NS == 0 and d % _L == 0

    n_chunks = -(-e // (_NW * _C))
    n_chunks = -(-n_chunks // (2 * _SG)) * (2 * _SG)
    e_pad = n_chunks * _NW * _C
    pad = e_pad - e

    src = a_in_edge_index[0].astype(jnp.int32)
    dst = a_in_edge_index[1].astype(jnp.int32)
    val = a_in_edge_values.astype(jnp.float32)
    if pad:
        src = jnp.concatenate([src, jnp.arange(pad, dtype=jnp.int32) % n])
        dst = jnp.concatenate([dst, jnp.zeros((pad,), jnp.int32)])
        val = jnp.concatenate([val, jnp.zeros((pad,), jnp.float32)])
    src = src.reshape(_NW, n_chunks, _C)
    src = src + (jnp.arange(_NW, dtype=jnp.int32)[:, None, None] // _NS) * n
    iv = jnp.stack([src, dst.reshape(_NW, n_chunks, _C)], axis=2)
    val = val.reshape(_NW, n_chunks, 1, _C)
    ego2 = jnp.concatenate([ego_embeddings, ego_embeddings], axis=0)

    partials = _sc_spmm(ego2, iv, val, n_chunks, n, d)
    return _tc_combine(ego_embeddings, partials, lin_W.T, lin_b, ln_g, ln_beta)
